# baseline (speedup 1.0000x reference)
import math

import jax
import jax.numpy as jnp
from jax import lax
from jax.experimental import pallas as pl
from jax.experimental.pallas import tpu as pltpu
from jax.experimental.pallas import tpu_sc as plsc

_SIZE = 100000
_N = 1024
_SMOOTHING = 0.1
_CONF = 1.0 - _SMOOTHING
_FILL = _SMOOTHING / (_SIZE - 2)
_C_ROW = (_SIZE - 2) * _FILL * math.log(_FILL) + _CONF * math.log(_CONF)

_NC = 2
_NS = 16
_LANES = 16
_NW = _NC * _NS
_B = _N // _NW

_BR = 64


def _sc_body(xr, tgt, out, tgt_v, idx_v, idx0_v, xt_v, x0_v, r_v, sem):
    wid = lax.axis_index("s") * _NC + lax.axis_index("c")
    base = wid * _B
    pltpu.sync_copy(tgt.at[pl.ds(base, _B)], tgt_v)
    for k in range(_B // _LANES):
        t = tgt_v[pl.ds(k * _LANES, _LANES)]
        i = base + k * _LANES + lax.iota(jnp.int32, _LANES)
        idx_v[pl.ds(k * _LANES, _LANES)] = i * _SIZE + t
        idx0_v[pl.ds(k * _LANES, _LANES)] = i * _SIZE
    pltpu.async_copy(xr.at[idx_v], xt_v, sem).wait()
    pltpu.async_copy(xr.at[idx0_v], x0_v, sem).wait()
    for k in range(_B // _LANES):
        t = tgt_v[pl.ds(k * _LANES, _LANES)]
        xt = xt_v[pl.ds(k * _LANES, _LANES)]
        x0 = x0_v[pl.ds(k * _LANES, _LANES)]
        r = jnp.where(
            t != 0,
            jnp.float32(_C_ROW)
            + jnp.float32(_FILL) * x0
            - jnp.float32(_CONF - _FILL) * xt,
            jnp.float32(0.0),
        )
        r_v[pl.ds(k * _LANES, _LANES)] = r
    pltpu.sync_copy(r_v, out.at[pl.ds(base, _B)])


def _sc_gather(xr, target):
    return pl.kernel(
        _sc_body,
        out_type=jax.ShapeDtypeStruct((_N,), jnp.float32),
        mesh=plsc.VectorSubcoreMesh(core_axis_name="c", subcore_axis_name="s"),
        scratch_types=[
            pltpu.VMEM((_B,), jnp.int32),
            pltpu.VMEM((_B,), jnp.int32),
            pltpu.VMEM((_B,), jnp.int32),
            pltpu.VMEM((_B,), jnp.float32),
            pltpu.VMEM((_B,), jnp.float32),
            pltpu.VMEM((_B,), jnp.float32),
            pltpu.SemaphoreType.DMA,
        ],
    )(xr, target)


def _tc_body(tgt_ref, r_ref, x_ref, out_ref, acc_ref):
    j = pl.program_id(0)

    @pl.when(j == 0)
    def _init():
        acc_ref[0] = jnp.float32(0.0)

    mask = tgt_ref[...] != 0
    acc_ref[0] += jnp.sum(jnp.where(mask, x_ref[...], jnp.float32(0.0)))

    @pl.when(j == pl.num_programs(0) - 1)
    def _finish():
        loss = jnp.sum(r_ref[...]) - jnp.float32(_FILL) * acc_ref[0]
        out_ref[...] = jnp.broadcast_to(loss, (1, 1))


def _tc_reduce(x, tgt2d, r2d):
    grid = (_N // _BR,)
    return pl.pallas_call(
        _tc_body,
        grid=grid,
        in_specs=[
            pl.BlockSpec((_BR, 1), lambda j: (j, 0)),
            pl.BlockSpec((_N, 1), lambda j: (0, 0)),
            pl.BlockSpec((_BR, _SIZE), lambda j: (j, 0)),
        ],
        out_specs=pl.BlockSpec((1, 1), lambda j: (0, 0)),
        out_shape=jax.ShapeDtypeStruct((1, 1), jnp.float32),
        scratch_shapes=[pltpu.SMEM((1,), jnp.float32)],
        compiler_params=pltpu.CompilerParams(
            dimension_semantics=("arbitrary",),
        ),
    )(tgt2d, r2d, x)


def kernel(x, target):
    xr = x.reshape(_N * _SIZE)
    r = _sc_gather(xr, target)
    out = _tc_reduce(x, target.reshape(_N, 1), r.reshape(_N, 1))
    return out[0, 0]
